# baseline (speedup 1.0000x reference)
import functools

import jax
import jax.numpy as jnp
import numpy as np
from jax import lax
from jax.experimental import pallas as pl
from jax.experimental.pallas import tpu as pltpu
from jax.experimental.pallas import tpu_sc as plsc

_FEAT_H = 50
_FEAT_W = 50
_IN_C = 256
_MID_C = 256
_N_ANCHOR = 9
_STRIDE = 16
_PRE_NMS = 2000
_POST_NMS = 300
_NMS_TH = 0.7
_MIN_SIZE = 16.0

_P = _FEAT_H * _FEAT_W
_PP = 52 * 52
_NB = _FEAT_H * _FEAT_W * _N_ANCHOR
_NBP = 176 * 128
_NC = 2048


def _make_anchors():
    base = 16.0
    ratios = [0.5, 1.0, 2.0]
    scales = [8.0, 16.0, 32.0]
    py = base / 2.0
    px = base / 2.0
    ab = np.zeros((9, 4), np.float32)
    for i, r in enumerate(ratios):
        for j, s in enumerate(scales):
            hh = base * s * np.sqrt(r)
            ww = base * s * np.sqrt(1.0 / r)
            k = i * 3 + j
            ab[k] = [py - hh / 2.0, px - ww / 2.0, py + hh / 2.0, px + ww / 2.0]
    sy = np.arange(0, _FEAT_H * _STRIDE, _STRIDE, dtype=np.float32)
    sx = np.arange(0, _FEAT_W * _STRIDE, _STRIDE, dtype=np.float32)
    sxg, syg = np.meshgrid(sx, sy)
    shift = np.stack([syg.ravel(), sxg.ravel(), syg.ravel(), sxg.ravel()], axis=1)
    return (shift[:, None, :] + ab[None, :, :]).reshape(-1, 4).astype(np.float32)


_ANCHORS_NP = _make_anchors()


def _head_body(xbt_ref, w9t_ref, b1_ref, wslt_ref, bsl_ref, out_ref):
    shifts = []
    for k in range(9):
        dy, dx = k // 3, k % 3
        off = 53 + (dy - 1) * 52 + (dx - 1)
        shifts.append(xbt_ref[off:off + _PP, :])
    x9t = jnp.concatenate(shifts, axis=1)
    acc = jnp.dot(x9t, w9t_ref[...], preferred_element_type=jnp.float32)
    feat = jnp.maximum(acc + b1_ref[:1, :], 0.0)
    out_ref[...] = jnp.dot(feat, wslt_ref[...],
                           preferred_element_type=jnp.float32) + bsl_ref[:1, :]


def _decode_body(l0_ref, l1_ref, loc_ref, anc_ref, msz_ref,
                 sc_ref, box_ref, dest_ref, tb_ref):
    l0 = l0_ref[...]
    l1 = l1_ref[...]
    m = jnp.maximum(l0, l1)
    e0 = jnp.exp(l0 - m)
    e1 = jnp.exp(l1 - m)
    fg = e1 / (e0 + e1)

    ay1 = anc_ref[0]
    ax1 = anc_ref[1]
    ay2 = anc_ref[2]
    ax2 = anc_ref[3]
    ah = ay2 - ay1
    aw = ax2 - ax1
    acy = ay1 + 0.5 * ah
    acx = ax1 + 0.5 * aw
    dy = loc_ref[0]
    dx = loc_ref[1]
    dh = loc_ref[2]
    dw = loc_ref[3]
    cy = dy * ah + acy
    cx = dx * aw + acx
    hh = jnp.exp(dh) * ah
    ww = jnp.exp(dw) * aw
    y1 = jnp.clip(cy - 0.5 * hh, 0.0, 800.0)
    x1 = jnp.clip(cx - 0.5 * ww, 0.0, 800.0)
    y2 = jnp.clip(cy + 0.5 * hh, 0.0, 800.0)
    x2 = jnp.clip(cx + 0.5 * ww, 0.0, 800.0)
    box_ref[0] = y1
    box_ref[1] = x1
    box_ref[2] = y2
    box_ref[3] = x2

    msz = msz_ref[0, 0]
    valid = ((y2 - y1) >= msz) & ((x2 - x1) >= msz)
    rows = lax.broadcasted_iota(jnp.int32, (176, 128), 0)
    cols = lax.broadcasted_iota(jnp.int32, (176, 128), 1)
    inb = (rows * 128 + cols) < _NB
    sc = jnp.where(valid & inb, fg, -jnp.inf)
    sc_ref[...] = sc

    bits = lax.bitcast_convert_type(sc, jnp.int32)
    keys = bits ^ ((bits >> 31) & jnp.int32(0x7FFFFFFF))

    def bis(_, lohi):
        lo, hi = lohi
        mid = lo + (hi - lo) // 2
        cnt = jnp.sum((keys >= mid).astype(jnp.int32))
        big = cnt >= _PRE_NMS
        return (jnp.where(big, mid, lo), jnp.where(big, hi, mid))

    v, _ = lax.fori_loop(0, 32, bis, (jnp.int32(-0x01000000), jnp.int32(0x40000000)))

    finite = sc > -jnp.inf
    sel = (keys >= v) & finite
    selef = sel.astype(jnp.float32)
    ltri = (lax.broadcasted_iota(jnp.int32, (128, 128), 0)
            <= lax.broadcasted_iota(jnp.int32, (128, 128), 1)).astype(jnp.float32)
    incl = jnp.dot(selef, ltri, preferred_element_type=jnp.float32)
    rowsum = incl[:, 127:128]
    stri = (lax.broadcasted_iota(jnp.int32, (176, 176), 1)
            < lax.broadcasted_iota(jnp.int32, (176, 176), 0)).astype(jnp.float32)
    rowoff = jnp.dot(stri, rowsum, preferred_element_type=jnp.float32)
    destf = rowoff + incl - selef
    dest = destf.astype(jnp.int32)
    dest_ref[...] = jnp.where(sel & (dest < _NC), dest, _NC)

    m0 = jnp.max(sc)
    em0 = sc == m0
    lane = lax.broadcasted_iota(jnp.int32, (1, 128), 1)
    t_y1 = jnp.sum(jnp.where(em0, y1, 0.0))
    t_x1 = jnp.sum(jnp.where(em0, x1, 0.0))
    t_y2 = jnp.sum(jnp.where(em0, y2, 0.0))
    t_x2 = jnp.sum(jnp.where(em0, x2, 0.0))
    tb_ref[...] = jnp.broadcast_to(
        jnp.where(lane == 0, t_y1,
        jnp.where(lane == 1, t_x1,
        jnp.where(lane == 2, t_y2,
        jnp.where(lane == 3, t_x2, 0.0)))), (8, 128))


_PAD_IDX = 22520
_TRASH = _NC
_CHUNK_ROWS = 16


def _compact_body(dest_ref, gidx_ref, sc_ref, b0_ref, b1_ref, b2_ref, b3_ref,
                  sidx_ref, csc_ref, cb0_ref, cb1_ref, cb2_ref, cb3_ref,
                  didx_v, ival_v, myidx_v, gbuf_v, init_v, sem):
    cid = lax.axis_index("c")
    sid = lax.axis_index("s")

    @pl.when(cid == 0)
    def _():
        for j in range(8):
            init_v[pl.ds(j * 16, 16)] = jnp.full((16,), _PAD_IDX, jnp.int32)
        pltpu.sync_copy(init_v, sidx_ref.at[pl.ds(sid * 128, 128)])

        @pl.when(sid == 0)
        def _():
            pltpu.sync_copy(init_v, sidx_ref.at[pl.ds(_NC, 128)])

        plsc.subcore_barrier()

        pltpu.sync_copy(dest_ref.at[pl.ds(sid * _CHUNK_ROWS, _CHUNK_ROWS), :],
                        didx_v)
        pltpu.sync_copy(gidx_ref.at[pl.ds(sid * _CHUNK_ROWS, _CHUNK_ROWS), :],
                        ival_v)
        for j in range(_CHUNK_ROWS):
            pltpu.async_copy(ival_v.at[j], sidx_ref.at[didx_v.at[j]], sem).wait()
        plsc.subcore_barrier()

        pltpu.sync_copy(sidx_ref.at[pl.ds(sid * 128, 128)], myidx_v)
        for src, dst in ((sc_ref, csc_ref), (b0_ref, cb0_ref),
                         (b1_ref, cb1_ref), (b2_ref, cb2_ref),
                         (b3_ref, cb3_ref)):
            pltpu.async_copy(src.at[myidx_v], gbuf_v, sem).wait()
            pltpu.sync_copy(gbuf_v, dst.at[pl.ds(sid * 128, 128)])


def _compact(dest, gidx, scf, b0, b1, b2, b3):
    mesh = plsc.VectorSubcoreMesh(core_axis_name="c", subcore_axis_name="s")
    f = pl.kernel(
        _compact_body,
        mesh=mesh,
        out_type=[
            jax.ShapeDtypeStruct((_NC + 128,), jnp.int32),
            jax.ShapeDtypeStruct((_NC,), jnp.float32),
            jax.ShapeDtypeStruct((_NC,), jnp.float32),
            jax.ShapeDtypeStruct((_NC,), jnp.float32),
            jax.ShapeDtypeStruct((_NC,), jnp.float32),
            jax.ShapeDtypeStruct((_NC,), jnp.float32),
        ],
        scratch_types=[
            pltpu.VMEM((_CHUNK_ROWS, 128), jnp.int32),
            pltpu.VMEM((_CHUNK_ROWS, 128), jnp.int32),
            pltpu.VMEM((128,), jnp.int32),
            pltpu.VMEM((128,), jnp.float32),
            pltpu.VMEM((128,), jnp.int32),
            pltpu.SemaphoreType.DMA,
        ],
    )
    return f(dest, gidx, scf, b0, b1, b2, b3)


def _nms_body(box_ref, sc_ref, tb_ref, out_ref):
    by1 = box_ref[0]
    bx1 = box_ref[1]
    by2 = box_ref[2]
    bx2 = box_ref[3]
    areas = (by2 - by1) * (bx2 - bx1)
    rows = lax.broadcasted_iota(jnp.int32, (16, 128), 0)
    cols = lax.broadcasted_iota(jnp.int32, (16, 128), 1)
    ii = rows * 128 + cols
    lane = lax.broadcasted_iota(jnp.int32, (1, 128), 1)

    def body(i, s):
        mval = jnp.max(s)
        exh = mval == -jnp.inf
        em = s == mval
        y1 = jnp.where(exh, tb_ref[0, 0], jnp.sum(jnp.where(em, by1, 0.0)))
        x1 = jnp.where(exh, tb_ref[0, 1], jnp.sum(jnp.where(em, bx1, 0.0)))
        y2 = jnp.where(exh, tb_ref[0, 2], jnp.sum(jnp.where(em, by2, 0.0)))
        x2 = jnp.where(exh, tb_ref[0, 3], jnp.sum(jnp.where(em, bx2, 0.0)))
        a = (y2 - y1) * (x2 - x1)
        iy1 = jnp.maximum(y1, by1)
        ix1 = jnp.maximum(x1, bx1)
        iy2 = jnp.minimum(y2, by2)
        ix2 = jnp.minimum(x2, bx2)
        inter = jnp.maximum(iy2 - iy1, 0.0) * jnp.maximum(ix2 - ix1, 0.0)
        iou = inter / (a + areas - inter + 1e-9)
        s = jnp.where((iou >= _NMS_TH) | em, -jnp.inf, s)
        row = jnp.where(lane == 0, y1,
              jnp.where(lane == 1, x1,
              jnp.where(lane == 2, y2,
              jnp.where(lane == 3, x2, 0.0))))
        out_ref[pl.ds(i, 1), :] = row
        return s

    lax.fori_loop(0, _POST_NMS, body, sc_ref[...])


def kernel(x, img_shape, W1, b1, Ws, bs, Wl, bl, scale):
    xp = jnp.pad(x[0], ((0, 0), (1, 1), (1, 1))).reshape(_IN_C, _PP)
    xbt = jnp.pad(jnp.transpose(xp, (1, 0)), ((53, 53), (0, 0)))
    w9t = jnp.transpose(W1, (2, 3, 1, 0)).reshape(9 * _IN_C, _MID_C)
    wsl = jnp.concatenate([Ws[:, :, 0, 0], Wl[:, :, 0, 0]], axis=0)
    wslt = jnp.transpose(jnp.pad(wsl, ((0, 10), (0, 0))), (1, 0))
    bsl = jnp.pad(jnp.concatenate([bs, bl]), (0, 10))

    sl_t = pl.pallas_call(
        _head_body,
        out_shape=jax.ShapeDtypeStruct((_PP, 64), jnp.float32),
    )(xbt, w9t, b1[None, :], wslt, bsl[None, :])
    sl = jnp.transpose(sl_t, (1, 0))

    sl_in = sl.reshape(64, 52, 52)[:, 1:51, 1:51].reshape(64, _P)
    score_flat = sl_in[:18]
    loc_flat = sl_in[18:54]
    rpn_score = jnp.transpose(score_flat, (1, 0)).reshape(1, _NB, 2)
    rpn_offset = loc_flat.reshape(1, _NB, 4)

    pad_n = _NBP - _NB
    l0 = jnp.pad(rpn_score[0, :, 0], (0, pad_n)).reshape(176, 128)
    l1 = jnp.pad(rpn_score[0, :, 1], (0, pad_n)).reshape(176, 128)
    loc4 = jnp.pad(jnp.transpose(rpn_offset[0], (1, 0)),
                   ((0, 0), (0, pad_n))).reshape(4, 176, 128)
    anc4 = jnp.asarray(
        np.pad(_ANCHORS_NP.T, ((0, 0), (0, pad_n))).reshape(4, 176, 128))
    msz = (jnp.float32(_MIN_SIZE) * scale).astype(jnp.float32).reshape(1, 1)

    sc, box4, dest, tb = pl.pallas_call(
        _decode_body,
        out_shape=[
            jax.ShapeDtypeStruct((176, 128), jnp.float32),
            jax.ShapeDtypeStruct((4, 176, 128), jnp.float32),
            jax.ShapeDtypeStruct((176, 128), jnp.int32),
            jax.ShapeDtypeStruct((8, 128), jnp.float32),
        ],
    )(l0, l1, loc4, anc4, msz)

    destp = jnp.pad(dest, ((0, 80), (0, 0)), constant_values=_NC)
    gidx = jnp.minimum(jnp.arange(256 * 128, dtype=jnp.int32),
                       _PAD_IDX).reshape(256, 128)
    scf = sc.reshape(_NBP)
    bf = box4.reshape(4, _NBP)
    _, csc, cb0, cb1, cb2, cb3 = _compact(
        destp, gidx, scf, bf[0], bf[1], bf[2], bf[3])
    bp = jnp.stack([cb0, cb1, cb2, cb3]).reshape(4, 16, 128)
    ts = csc.reshape(16, 128)

    rois_pad = pl.pallas_call(
        _nms_body,
        out_shape=jax.ShapeDtypeStruct((304, 128), jnp.float32),
    )(bp, ts, tb)
    rois = rois_pad[:_POST_NMS, :4]

    roi_indices = jnp.zeros((_POST_NMS,), jnp.int32)
    anchors = jnp.asarray(_ANCHORS_NP)[None]
    return (rpn_offset, rpn_score, rois, roi_indices, anchors)
